# baseline (speedup 1.0000x reference)
import functools

import jax
import jax.numpy as jnp
from jax import lax
from jax.experimental import pallas as pl
from jax.experimental.pallas import tpu as pltpu
from jax.experimental.pallas import tpu_sc as plsc

N = 10000
E = 320000
D_IN = 128
D_H = 128
D_OUT = 64

NC = 2
NS = 16
NW = NC * NS
LANES = 16

EPT = E // NW
K = 80
CPT = EPT // K
DR = 1000
ZR = 40

assert E % NW == 0 and EPT % K == 0 and K % 8 == 0 and K % LANES == 0
assert CPT % 2 == 1
assert N % DR == 0 and N // DR <= NS and DR % ZR == 0
assert DR % 8 == 0 and ZR % 8 == 0 and N % LANES == 0


def _sc_mesh():
    return plsc.VectorSubcoreMesh(core_axis_name="c", subcore_axis_name="s",
                                  num_cores=NC, num_subcores=NS)


_ZCH = 2000
assert N % _ZCH == 0 and _ZCH % LANES == 0 and _ZCH % 8 == 0


_DB = 5
assert CPT % _DB == 0


def _deg_body(col_hbm, ew_hbm, o0_hbm, col_v, ew_v, zero_v, deg_sh, dsem):
    cid = lax.axis_index("c")
    sid = lax.axis_index("s")

    zb = jnp.zeros((LANES,), jnp.float32)

    def zbody(i, carry):
        zero_v[pl.ds(i * LANES, LANES)] = zb
        return carry

    lax.fori_loop(0, _ZCH // LANES, zbody, 0)

    @pl.when(sid < N // _ZCH)
    def _():
        pltpu.sync_copy(zero_v, deg_sh.at[pl.ds(sid * _ZCH, _ZCH)])

    plsc.subcore_barrier()

    cbase = (cid * NS + sid) * CPT
    pltpu.sync_copy(col_hbm.at[pl.ds(cbase, CPT)], col_v)
    pltpu.sync_copy(ew_hbm.at[pl.ds(cbase, CPT)], ew_v)

    def batch(i, carry):
        for b in range(_DB):
            ci = i * _DB + b
            pltpu.async_copy(ew_v.at[ci], deg_sh.at[col_v.at[ci]], dsem,
                             add=True)
        for b in range(_DB):
            pltpu.make_async_copy(ew_hbm.at[0], ew_v.at[0], dsem).wait()
        return carry

    lax.fori_loop(0, CPT // _DB, batch, 0)
    plsc.subcore_barrier()

    @pl.when(sid < N // _ZCH)
    def _():
        pltpu.sync_copy(deg_sh.at[pl.ds(sid * _ZCH, _ZCH)], zero_v)
        pltpu.sync_copy(zero_v, o0_hbm.at[pl.ds(cid * N + sid * _ZCH, _ZCH)])


_deg_kernel = functools.partial(
    pl.kernel,
    out_type=jax.ShapeDtypeStruct((NC * N,), jnp.float32),
    mesh=_sc_mesh(),
    scratch_types=[
        pltpu.VMEM((CPT, K), jnp.int32),
        pltpu.VMEM((CPT, K), jnp.float32),
        pltpu.VMEM((_ZCH,), jnp.float32),
        pltpu.VMEM_SHARED((N,), jnp.float32),
        pltpu.SemaphoreType.DMA,
    ],
    compiler_params=pltpu.CompilerParams(use_tc_tiling_on_sc=False),
)(_deg_body)


_GDN = lax.GatherDimensionNumbers(
    offset_dims=(), collapsed_slice_dims=(0,), start_index_map=(0,))


def _bcast_lane(vec, i):
    idx = jnp.full((LANES, 1), i, jnp.int32)
    return lax.gather(vec, idx, _GDN, (1,),
                      mode=lax.GatherScatterMode.PROMISE_IN_BOUNDS)


def _make_agg(D):
    def body(y_hbm, rc_hbm, ew_hbm, out_hbm,
             rc_v, ew_v, row0, row1, colb0, colb1, buf0, buf1, zero_v, acc_sh,
             gsem0, gsem1, ssem0, ssem1):
        cid = lax.axis_index("c")
        sid = lax.axis_index("s")

        cbase = (cid * NS + sid) * CPT
        pltpu.async_copy(rc_hbm.at[pl.ds(cbase, CPT)], rc_v, gsem0)
        pltpu.async_copy(ew_hbm.at[pl.ds(cbase, CPT)], ew_v, gsem1)

        def unpack(ci, rowbuf, colbuf):
            for g in range(K // LANES):
                sl = pl.ds(g * LANES, LANES)
                v = rc_v[ci, sl]
                rowbuf[sl] = lax.shift_right_logical(v, 16)
                colbuf[sl] = lax.bitwise_and(v, 0xFFFF)

        def drain_rows(sem, nr, buf):
            pltpu.make_async_copy(
                y_hbm.at[pl.ds(0, nr)], buf.at[pl.ds(0, nr)], sem).wait()

        zb = jnp.zeros((LANES,), jnp.float32)

        def zrow(i, carry):
            for c2 in range(D // LANES):
                zero_v[i, pl.ds(c2 * LANES, LANES)] = zb
            return carry

        lax.fori_loop(0, ZR, zrow, 0)

        pltpu.make_async_copy(rc_hbm.at[pl.ds(cbase, CPT)], rc_v, gsem0).wait()
        pltpu.make_async_copy(ew_hbm.at[pl.ds(cbase, CPT)], ew_v, gsem1).wait()
        unpack(0, row0, colb0)
        pltpu.async_copy(y_hbm.at[row0], buf0, gsem0)

        RZT = N // NS
        zsizes = [ZR] * (RZT // ZR) + ([RZT % ZR] if RZT % ZR else [])
        for r, sz in enumerate(zsizes):
            pltpu.async_copy(zero_v.at[pl.ds(0, sz)],
                             acc_sh.at[pl.ds(sid * RZT + r * ZR, sz)], ssem1)
        for sz in zsizes:
            drain_rows(ssem1, sz, zero_v)

        plsc.subcore_barrier()

        def scale(buf, ci):
            def scale16(g, c2):
                w16 = ew_v[ci, pl.ds(g * LANES, LANES)]
                for i in range(LANES):
                    w = _bcast_lane(w16, i)
                    j = g * LANES + i
                    for d in range(D // LANES):
                        sl = pl.ds(d * LANES, LANES)
                        buf[j, sl] = buf[j, sl] * w
                return c2

            lax.fori_loop(0, K // LANES, scale16, 0)

        def drain(sem, buf):
            pltpu.make_async_copy(y_hbm.at[pl.ds(0, K)], buf, sem).wait()

        def chunk_pair(i, carry):
            c0 = 2 * i
            c1 = c0 + 1

            @pl.when(i > 0)
            def _():
                drain(ssem1, buf1)

            unpack(c1, row1, colb1)
            pltpu.async_copy(y_hbm.at[row1], buf1, gsem1, priority=1)
            drain(gsem0, buf0)
            scale(buf0, c0)
            pltpu.async_copy(buf0, acc_sh.at[colb0], ssem0, add=True)
            drain(gsem1, buf1)
            scale(buf1, c1)
            drain(ssem0, buf0)
            unpack(c0 + 2, row0, colb0)
            pltpu.async_copy(y_hbm.at[row0], buf0, gsem0)
            pltpu.async_copy(buf1, acc_sh.at[colb1], ssem1, add=True)
            return carry

        lax.fori_loop(0, (CPT - 1) // 2, chunk_pair, 0)
        drain(ssem1, buf1)
        drain(gsem0, buf0)
        scale(buf0, CPT - 1)
        pltpu.sync_copy(buf0, acc_sh.at[colb0], add=True)
        plsc.subcore_barrier()

        dsizes = [K] * (RZT // K) + ([RZT % K] if RZT % K else [])
        for r, sz in enumerate(dsizes):
            b = buf0 if r % 2 == 0 else buf1
            ss = ssem0 if r % 2 == 0 else ssem1
            off = sid * RZT + r * K
            if r >= 2:
                drain_rows(ss, dsizes[r - 2], b)
            pltpu.sync_copy(acc_sh.at[pl.ds(off, sz)], b.at[pl.ds(0, sz)])
            pltpu.async_copy(b.at[pl.ds(0, sz)],
                             out_hbm.at[cid, pl.ds(off, sz)], ss)
        for r in (len(dsizes) - 2, len(dsizes) - 1):
            b = buf0 if r % 2 == 0 else buf1
            ss = ssem0 if r % 2 == 0 else ssem1
            drain_rows(ss, dsizes[r], b)

    return functools.partial(
        pl.kernel,
        out_type=jax.ShapeDtypeStruct((NC, N, D), jnp.float32),
        mesh=_sc_mesh(),
        scratch_types=[
            pltpu.VMEM((CPT, K), jnp.int32),
            pltpu.VMEM((CPT, K), jnp.float32),
            pltpu.VMEM((K,), jnp.int32),
            pltpu.VMEM((K,), jnp.int32),
            pltpu.VMEM((K,), jnp.int32),
            pltpu.VMEM((K,), jnp.int32),
            pltpu.VMEM((K, D), jnp.float32),
            pltpu.VMEM((K, D), jnp.float32),
            pltpu.VMEM((ZR, D), jnp.float32),
            pltpu.VMEM_SHARED((N, D), jnp.float32),
            pltpu.SemaphoreType.DMA,
            pltpu.SemaphoreType.DMA,
            pltpu.SemaphoreType.DMA,
            pltpu.SemaphoreType.DMA,
        ],
        compiler_params=pltpu.CompilerParams(use_tc_tiling_on_sc=False),
    )(body)


_agg_h = _make_agg(D_H)


_R = 2000
assert N % _R == 0


def _dis_body(degp_ref, dis_ref):
    deg = jnp.sum(degp_ref[...], axis=0) + 1.0
    dis_ref[...] = jnp.where(deg > 0, lax.rsqrt(deg), 0.0)[:, None]


def _dis_kernel(degp):
    return pl.pallas_call(
        _dis_body,
        out_shape=jax.ShapeDtypeStruct((N, 1), jnp.float32),
    )(degp)


def _head_body(dis_ref, x_ref, w1_ref, y1_ref):
    xw = jnp.dot(x_ref[...], w1_ref[...], preferred_element_type=jnp.float32)
    y1_ref[...] = xw * dis_ref[...]


def _head(dis, x, W1):
    return pl.pallas_call(
        _head_body,
        grid=(N // _R,),
        in_specs=[
            pl.BlockSpec((_R, 1), lambda b: (b, 0)),
            pl.BlockSpec((_R, D_IN), lambda b: (b, 0)),
            pl.BlockSpec((D_IN, D_H), lambda b: (0, 0)),
        ],
        out_specs=pl.BlockSpec((_R, D_H), lambda b: (b, 0)),
        out_shape=jax.ShapeDtypeStruct((N, D_H), jnp.float32),
    )(dis, x, W1)


def _mid_body(acc_ref, y1_ref, dis_ref, b1_ref, w2_ref, y2_ref):
    dis = dis_ref[...]
    t = (acc_ref[0] + acc_ref[1] + y1_ref[...]) * dis + b1_ref[...][None, :]
    h = jnp.maximum(t, 0.0)
    hw = jnp.dot(h, w2_ref[...], preferred_element_type=jnp.float32)
    y2_ref[...] = jnp.concatenate(
        [hw * dis, jnp.zeros_like(hw)], axis=1)


def _mid(acc1, y1, dis, b1, W2):
    return pl.pallas_call(
        _mid_body,
        grid=(N // _R,),
        in_specs=[
            pl.BlockSpec((NC, _R, D_H), lambda b: (0, b, 0)),
            pl.BlockSpec((_R, D_H), lambda b: (b, 0)),
            pl.BlockSpec((_R, 1), lambda b: (b, 0)),
            pl.BlockSpec((D_H,), lambda b: (0,)),
            pl.BlockSpec((D_H, D_OUT), lambda b: (0, 0)),
        ],
        out_specs=pl.BlockSpec((_R, 2 * D_OUT), lambda b: (b, 0)),
        out_shape=jax.ShapeDtypeStruct((N, 2 * D_OUT), jnp.float32),
    )(acc1, y1, dis, b1, W2)


def _tail_body(acc_ref, y2_ref, dis_ref, b2_ref, o_ref):
    dis = dis_ref[...]
    s = acc_ref[0] + acc_ref[1] + y2_ref[...]
    z = s[:, :D_OUT] * dis + b2_ref[...][None, :]
    m = jnp.max(z, axis=1, keepdims=True)
    ez = jnp.exp(z - m)
    o_ref[...] = z - m - jnp.log(jnp.sum(ez, axis=1, keepdims=True))


def _tail(acc2, y2, dis, b2):
    return pl.pallas_call(
        _tail_body,
        grid=(N // _R,),
        in_specs=[
            pl.BlockSpec((NC, _R, 2 * D_OUT), lambda b: (0, b, 0)),
            pl.BlockSpec((_R, 2 * D_OUT), lambda b: (b, 0)),
            pl.BlockSpec((_R, 1), lambda b: (b, 0)),
            pl.BlockSpec((D_OUT,), lambda b: (0,)),
        ],
        out_specs=pl.BlockSpec((_R, D_OUT), lambda b: (b, 0)),
        out_shape=jax.ShapeDtypeStruct((N, D_OUT), jnp.float32),
    )(acc2, y2, dis, b2)


def kernel(x, edge_index, edge_attr, W1, b1, W2, b2):
    col2 = edge_index[1].reshape(E // K, K)
    ew2 = edge_attr.reshape(E // K, K)
    rc2 = (edge_index[0] * 65536 + edge_index[1]).reshape(E // K, K)
    degp = _deg_kernel(col2, ew2).reshape(NC, N)
    dis = _dis_kernel(degp)
    y1 = _head(dis, x, W1)
    acc1 = _agg_h(y1, rc2, ew2)
    y2 = _mid(acc1, y1, dis, b1, W2)
    acc2 = _agg_h(y2, rc2, ew2)
    return _tail(acc2, y2, dis, b2)


# speedup vs baseline: 1.0014x; 1.0014x over previous
import functools

import jax
import jax.numpy as jnp
from jax import lax
from jax.experimental import pallas as pl
from jax.experimental.pallas import tpu as pltpu
from jax.experimental.pallas import tpu_sc as plsc

N = 10000
E = 320000
D_IN = 128
D_H = 128
D_OUT = 64

NC = 2
NS = 16
NW = NC * NS
LANES = 16

EPT = E // NW
K = 80
CPT = EPT // K
ZR = 40

assert E % NW == 0 and EPT % K == 0 and K % 8 == 0 and K % LANES == 0
assert CPT % 2 == 1
assert ZR % 8 == 0 and N % LANES == 0 and N % NS == 0


def _sc_mesh():
    return plsc.VectorSubcoreMesh(core_axis_name="c", subcore_axis_name="s",
                                  num_cores=NC, num_subcores=NS)


_ZCH = 2000
assert N % _ZCH == 0 and _ZCH % LANES == 0 and _ZCH % 8 == 0


_DB = 5
assert CPT % _DB == 0


def _deg_body(col_hbm, ew_hbm, o0_hbm, col_v, ew_v, zero_v, deg_sh, dsem):
    cid = lax.axis_index("c")
    sid = lax.axis_index("s")

    zb = jnp.zeros((LANES,), jnp.float32)

    def zbody(i, carry):
        zero_v[pl.ds(i * LANES, LANES)] = zb
        return carry

    lax.fori_loop(0, _ZCH // LANES, zbody, 0)

    @pl.when(sid < N // _ZCH)
    def _():
        pltpu.sync_copy(zero_v, deg_sh.at[pl.ds(sid * _ZCH, _ZCH)])

    plsc.subcore_barrier()

    cbase = (cid * NS + sid) * CPT
    pltpu.sync_copy(col_hbm.at[pl.ds(cbase, CPT)], col_v)
    pltpu.sync_copy(ew_hbm.at[pl.ds(cbase, CPT)], ew_v)

    def batch(i, carry):
        for b in range(_DB):
            ci = i * _DB + b
            pltpu.async_copy(ew_v.at[ci], deg_sh.at[col_v.at[ci]], dsem,
                             add=True)
        for b in range(_DB):
            pltpu.make_async_copy(ew_hbm.at[0], ew_v.at[0], dsem).wait()
        return carry

    lax.fori_loop(0, CPT // _DB, batch, 0)
    plsc.subcore_barrier()

    @pl.when(sid < N // _ZCH)
    def _():
        pltpu.sync_copy(deg_sh.at[pl.ds(sid * _ZCH, _ZCH)], zero_v)
        pltpu.sync_copy(zero_v, o0_hbm.at[pl.ds(cid * N + sid * _ZCH, _ZCH)])


_deg_kernel = functools.partial(
    pl.kernel,
    out_type=jax.ShapeDtypeStruct((NC * N,), jnp.float32),
    mesh=_sc_mesh(),
    scratch_types=[
        pltpu.VMEM((CPT, K), jnp.int32),
        pltpu.VMEM((CPT, K), jnp.float32),
        pltpu.VMEM((_ZCH,), jnp.float32),
        pltpu.VMEM_SHARED((N,), jnp.float32),
        pltpu.SemaphoreType.DMA,
    ],
    compiler_params=pltpu.CompilerParams(use_tc_tiling_on_sc=False),
)(_deg_body)


_GDN = lax.GatherDimensionNumbers(
    offset_dims=(), collapsed_slice_dims=(0,), start_index_map=(0,))


def _bcast_lane(vec, i):
    idx = jnp.full((LANES, 1), i, jnp.int32)
    return lax.gather(vec, idx, _GDN, (1,),
                      mode=lax.GatherScatterMode.PROMISE_IN_BOUNDS)


def _make_agg(D):
    def body(y_hbm, rc_hbm, ew_hbm, out_hbm,
             rc_v, ew_v, row0, row1, colb0, colb1, buf0, buf1, zero_v, acc_sh,
             gsem0, gsem1, ssem0, ssem1):
        cid = lax.axis_index("c")
        sid = lax.axis_index("s")

        cbase = (cid * NS + sid) * CPT
        pltpu.async_copy(rc_hbm.at[pl.ds(cbase, CPT)], rc_v, gsem0)
        pltpu.async_copy(ew_hbm.at[pl.ds(cbase, CPT)], ew_v, gsem1)

        def unpack(ci, rowbuf, colbuf):
            for g in range(K // LANES):
                sl = pl.ds(g * LANES, LANES)
                v = rc_v[ci, sl]
                rowbuf[sl] = lax.shift_right_logical(v, 16)
                colbuf[sl] = lax.bitwise_and(v, 0xFFFF)

        def drain_rows(sem, nr, buf):
            pltpu.make_async_copy(
                y_hbm.at[pl.ds(0, nr)], buf.at[pl.ds(0, nr)], sem).wait()

        zb = jnp.zeros((LANES,), jnp.float32)

        def zrow(i, carry):
            for c2 in range(D // LANES):
                zero_v[i, pl.ds(c2 * LANES, LANES)] = zb
            return carry

        lax.fori_loop(0, ZR, zrow, 0)

        pltpu.make_async_copy(rc_hbm.at[pl.ds(cbase, CPT)], rc_v, gsem0).wait()
        pltpu.make_async_copy(ew_hbm.at[pl.ds(cbase, CPT)], ew_v, gsem1).wait()
        unpack(0, row0, colb0)
        pltpu.async_copy(y_hbm.at[row0], buf0, gsem0)

        RZT = N // NS
        zsizes = [ZR] * (RZT // ZR) + ([RZT % ZR] if RZT % ZR else [])
        for r, sz in enumerate(zsizes):
            pltpu.async_copy(zero_v.at[pl.ds(0, sz)],
                             acc_sh.at[pl.ds(sid * RZT + r * ZR, sz)], ssem1)
        for sz in zsizes:
            drain_rows(ssem1, sz, zero_v)

        plsc.subcore_barrier()

        def scale(buf, ci):
            def scale16(g, c2):
                w16 = ew_v[ci, pl.ds(g * LANES, LANES)]
                for i in range(LANES):
                    w = _bcast_lane(w16, i)
                    j = g * LANES + i
                    for d in range(D // LANES):
                        sl = pl.ds(d * LANES, LANES)
                        buf[j, sl] = buf[j, sl] * w
                return c2

            lax.fori_loop(0, K // LANES, scale16, 0)

        def drain(sem, buf):
            pltpu.make_async_copy(y_hbm.at[pl.ds(0, K)], buf, sem).wait()

        def chunk_pair(i, carry):
            c0 = 2 * i
            c1 = c0 + 1

            @pl.when(i > 0)
            def _():
                drain(ssem1, buf1)

            unpack(c1, row1, colb1)
            pltpu.async_copy(y_hbm.at[row1], buf1, gsem1)
            drain(gsem0, buf0)
            scale(buf0, c0)
            pltpu.async_copy(buf0, acc_sh.at[colb0], ssem0, add=True)
            drain(gsem1, buf1)
            scale(buf1, c1)
            drain(ssem0, buf0)
            unpack(c0 + 2, row0, colb0)
            pltpu.async_copy(y_hbm.at[row0], buf0, gsem0)
            pltpu.async_copy(buf1, acc_sh.at[colb1], ssem1, add=True)
            return carry

        lax.fori_loop(0, (CPT - 1) // 2, chunk_pair, 0)
        drain(ssem1, buf1)
        drain(gsem0, buf0)
        scale(buf0, CPT - 1)
        pltpu.sync_copy(buf0, acc_sh.at[colb0], add=True)
        plsc.subcore_barrier()

        dsizes = [K] * (RZT // K) + ([RZT % K] if RZT % K else [])
        for r, sz in enumerate(dsizes):
            b = buf0 if r % 2 == 0 else buf1
            ss = ssem0 if r % 2 == 0 else ssem1
            off = sid * RZT + r * K
            if r >= 2:
                drain_rows(ss, dsizes[r - 2], b)
            pltpu.sync_copy(acc_sh.at[pl.ds(off, sz)], b.at[pl.ds(0, sz)])
            pltpu.async_copy(b.at[pl.ds(0, sz)],
                             out_hbm.at[cid, pl.ds(off, sz)], ss)
        for r in (len(dsizes) - 2, len(dsizes) - 1):
            b = buf0 if r % 2 == 0 else buf1
            ss = ssem0 if r % 2 == 0 else ssem1
            drain_rows(ss, dsizes[r], b)

    return functools.partial(
        pl.kernel,
        out_type=jax.ShapeDtypeStruct((NC, N, D), jnp.float32),
        mesh=_sc_mesh(),
        scratch_types=[
            pltpu.VMEM((CPT, K), jnp.int32),
            pltpu.VMEM((CPT, K), jnp.float32),
            pltpu.VMEM((K,), jnp.int32),
            pltpu.VMEM((K,), jnp.int32),
            pltpu.VMEM((K,), jnp.int32),
            pltpu.VMEM((K,), jnp.int32),
            pltpu.VMEM((K, D), jnp.float32),
            pltpu.VMEM((K, D), jnp.float32),
            pltpu.VMEM((ZR, D), jnp.float32),
            pltpu.VMEM_SHARED((N, D), jnp.float32),
            pltpu.SemaphoreType.DMA,
            pltpu.SemaphoreType.DMA,
            pltpu.SemaphoreType.DMA,
            pltpu.SemaphoreType.DMA,
        ],
        compiler_params=pltpu.CompilerParams(use_tc_tiling_on_sc=False),
    )(body)


_agg_h = _make_agg(D_H)


_R = 2000
assert N % _R == 0


def _dis_body(degp_ref, dis_ref):
    deg = jnp.sum(degp_ref[...], axis=0) + 1.0
    dis_ref[...] = jnp.where(deg > 0, lax.rsqrt(deg), 0.0)[:, None]


def _dis_kernel(degp):
    return pl.pallas_call(
        _dis_body,
        out_shape=jax.ShapeDtypeStruct((N, 1), jnp.float32),
    )(degp)


def _head_body(dis_ref, x_ref, w1_ref, y1_ref):
    xw = jnp.dot(x_ref[...], w1_ref[...], preferred_element_type=jnp.float32)
    y1_ref[...] = xw * dis_ref[...]


def _head(dis, x, W1):
    return pl.pallas_call(
        _head_body,
        grid=(N // _R,),
        in_specs=[
            pl.BlockSpec((_R, 1), lambda b: (b, 0)),
            pl.BlockSpec((_R, D_IN), lambda b: (b, 0)),
            pl.BlockSpec((D_IN, D_H), lambda b: (0, 0)),
        ],
        out_specs=pl.BlockSpec((_R, D_H), lambda b: (b, 0)),
        out_shape=jax.ShapeDtypeStruct((N, D_H), jnp.float32),
    )(dis, x, W1)


def _mid_body(acc_ref, y1_ref, dis_ref, b1_ref, w2_ref, y2_ref):
    dis = dis_ref[...]
    t = (acc_ref[0] + acc_ref[1] + y1_ref[...]) * dis + b1_ref[...][None, :]
    h = jnp.maximum(t, 0.0)
    hw = jnp.dot(h, w2_ref[...], preferred_element_type=jnp.float32)
    y2_ref[...] = jnp.concatenate(
        [hw * dis, jnp.zeros_like(hw)], axis=1)


def _mid(acc1, y1, dis, b1, W2):
    return pl.pallas_call(
        _mid_body,
        grid=(N // _R,),
        in_specs=[
            pl.BlockSpec((NC, _R, D_H), lambda b: (0, b, 0)),
            pl.BlockSpec((_R, D_H), lambda b: (b, 0)),
            pl.BlockSpec((_R, 1), lambda b: (b, 0)),
            pl.BlockSpec((D_H,), lambda b: (0,)),
            pl.BlockSpec((D_H, D_OUT), lambda b: (0, 0)),
        ],
        out_specs=pl.BlockSpec((_R, 2 * D_OUT), lambda b: (b, 0)),
        out_shape=jax.ShapeDtypeStruct((N, 2 * D_OUT), jnp.float32),
    )(acc1, y1, dis, b1, W2)


def _tail_body(acc_ref, y2_ref, dis_ref, b2_ref, o_ref):
    dis = dis_ref[...]
    s = acc_ref[0] + acc_ref[1] + y2_ref[...]
    z = s[:, :D_OUT] * dis + b2_ref[...][None, :]
    m = jnp.max(z, axis=1, keepdims=True)
    ez = jnp.exp(z - m)
    o_ref[...] = z - m - jnp.log(jnp.sum(ez, axis=1, keepdims=True))


def _tail(acc2, y2, dis, b2):
    return pl.pallas_call(
        _tail_body,
        grid=(N // _R,),
        in_specs=[
            pl.BlockSpec((NC, _R, 2 * D_OUT), lambda b: (0, b, 0)),
            pl.BlockSpec((_R, 2 * D_OUT), lambda b: (b, 0)),
            pl.BlockSpec((_R, 1), lambda b: (b, 0)),
            pl.BlockSpec((D_OUT,), lambda b: (0,)),
        ],
        out_specs=pl.BlockSpec((_R, D_OUT), lambda b: (b, 0)),
        out_shape=jax.ShapeDtypeStruct((N, D_OUT), jnp.float32),
    )(acc2, y2, dis, b2)


def kernel(x, edge_index, edge_attr, W1, b1, W2, b2):
    col2 = edge_index[1].reshape(E // K, K)
    ew2 = edge_attr.reshape(E // K, K)
    rc2 = (edge_index[0] * 65536 + edge_index[1]).reshape(E // K, K)
    degp = _deg_kernel(col2, ew2).reshape(NC, N)
    dis = _dis_kernel(degp)
    y1 = _head(dis, x, W1)
    acc1 = _agg_h(y1, rc2, ew2)
    y2 = _mid(acc1, y1, dis, b1, W2)
    acc2 = _agg_h(y2, rc2, ew2)
    return _tail(acc2, y2, dis, b2)
